# baseline (speedup 1.0000x reference)
import jax
import jax.numpy as jnp
from jax import lax
from jax.experimental import pallas as pl
from jax.experimental.pallas import tpu as pltpu

N = 35
F1 = 140
F2 = 280
NCHUNK = 4
CH = F1 // NCHUNK


def _gclstm_fused_kernel(adj_ref, w1_ref, wc1_ref, wc2_ref, fcwt_ref,
                         w2_hbm, out_ref, w2_vmem, sems):
    cps = [
        pltpu.make_async_copy(
            w2_hbm.at[pl.ds(i * CH, CH)], w2_vmem.at[pl.ds(i * CH, CH)],
            sems.at[i])
        for i in range(NCHUNK)
    ]
    for cp in cps:
        cp.start()

    X = adj_ref[...]

    def layer(X, w_ref, wc_ref):
        gi = jnp.dot(X, w_ref[:, 0, :], preferred_element_type=jnp.float32)
        gt = jnp.dot(X, w_ref[:, 2, :], preferred_element_type=jnp.float32)
        go = jnp.dot(X, w_ref[:, 3, :], preferred_element_type=jnp.float32)
        I = jax.nn.sigmoid(gi)
        T = jnp.tanh(gt)
        C = I * T
        O = jax.nn.sigmoid(go + wc_ref[2] * C)
        return jax.nn.relu(O * jnp.tanh(C))

    H1 = layer(X, w1_ref, wc1_ref)
    for cp in cps:
        cp.wait()
    H2 = layer(H1, w2_vmem, wc2_ref)
    Y = lax.dot_general(H2, fcwt_ref[...], (((1,), (1,)), ((), ())),
                        preferred_element_type=jnp.float32)
    out_ref[...] = jax.nn.relu(Y)


def kernel(adj_matrix, c1_Wx, c1_b, c1_wc, c1_chebW, c1_chebb,
           c2_Wx, c2_b, c2_wc, c2_chebW, c2_chebb, fc1_W, fc1_b):
    del c1_chebW, c2_chebW, c1_b, c1_chebb, c2_b, c2_chebb, fc1_b
    w1t = jnp.transpose(c1_Wx, (1, 0, 2))
    w2t = jnp.transpose(c2_Wx, (1, 0, 2))
    fcwt = fc1_W.T
    vmem = pl.BlockSpec(memory_space=pltpu.MemorySpace.VMEM)
    return pl.pallas_call(
        _gclstm_fused_kernel,
        out_shape=jax.ShapeDtypeStruct((N, N), jnp.float32),
        in_specs=[vmem, vmem, vmem, vmem, vmem,
                  pl.BlockSpec(memory_space=pltpu.MemorySpace.HBM)],
        scratch_shapes=[pltpu.VMEM((F1, 4, F2), jnp.float32),
                        pltpu.SemaphoreType.DMA((NCHUNK,))],
    )(adj_matrix, w1t, c1_wc, c2_wc, fcwt, w2t)
